# baseline (speedup 1.0000x reference)
import functools

import jax
import jax.numpy as jnp
import numpy as np
from jax.experimental import pallas as pl
from jax.experimental.pallas import tpu as pltpu

K_FRAC = 0.1
KEY_NEG_INF = np.int32(-2139095041)
KEY_POS_INF = np.int32(0x7F800000)
ZONE_M = 2
CAP_A = 10
CAP_B = 34
T0 = 1.2815516


def _f2k(f):
    b = jax.lax.bitcast_convert_type(f, jnp.int32)
    return b ^ (jax.lax.shift_right_arithmetic(b, 31) & np.int32(0x7FFFFFFF))


def _k2f(t):
    tc = jnp.maximum(t, KEY_NEG_INF)
    fbits = jnp.where(tc < 0, tc ^ np.int32(0x7FFFFFFF), tc)
    return jax.lax.bitcast_convert_type(fbits, jnp.float32)


def _topk_mask_block(x_ref, o_ref, lo_ref, hi_ref, clo_ref, chi_ref,
                     bkey_ref, j_ref, done_ref, t_ref, *, k, n):
    x = x_ref[...]
    kf = np.int32(k)
    rows = x.shape[0]
    zeros = jnp.zeros((rows, 1), jnp.int32)

    lo_ref[...] = zeros + KEY_NEG_INF
    hi_ref[...] = zeros + KEY_POS_INF
    clo_ref[...] = zeros + np.int32(n)
    chi_ref[...] = zeros
    bkey_ref[...] = zeros
    j_ref[...] = zeros
    done_ref[...] = zeros
    t_ref[...] = jnp.zeros((rows, 1), jnp.float32) + np.float32(T0)

    def count_ge(tf):
        return jnp.sum((x >= tf).astype(jnp.int32), axis=1, keepdims=True)

    def probe_update(t_key):
        tf = _k2f(t_key)
        c = count_ge(tf)
        done = done_ref[...] > 0
        ge = c >= kf
        upd = jnp.logical_not(done)
        lo2 = jnp.where(upd & ge, t_key, lo_ref[...])
        clo2 = jnp.where(upd & ge, c, clo_ref[...])
        hi2 = jnp.where(upd & ~ge, t_key, hi_ref[...])
        chi2 = jnp.where(upd & ~ge, c, chi_ref[...])
        zone = upd & ge & (c <= kf + np.int32(ZONE_M))
        narrow = upd & ~zone & (hi2 <= lo2 + np.int32(1))
        bkey_ref[...] = jnp.where(zone, t_key,
                                  jnp.where(narrow, lo2, bkey_ref[...]))
        j_ref[...] = jnp.where(zone, c - kf,
                               jnp.where(narrow, np.int32(0), j_ref[...]))
        done2 = done | zone | narrow
        lo_ref[...] = lo2
        hi_ref[...] = hi2
        clo_ref[...] = clo2
        chi_ref[...] = chi2
        done_ref[...] = done2.astype(jnp.int32)
        notdone = jnp.sum(jnp.logical_not(done2).astype(jnp.int32))
        return c, tf, notdone

    def a_cond(carry):
        i, notdone = carry
        return jnp.logical_and(i < CAP_A, notdone > 0)

    def a_body(carry):
        i, _ = carry
        lo, hi = lo_ref[...], hi_ref[...]
        c_lo, c_hi = clo_ref[...], chi_ref[...]
        has_both = (lo > KEY_NEG_INF) & (hi < KEY_POS_INF)
        frac = (c_lo - (kf + np.int32(1))).astype(jnp.float32) / jnp.maximum(
            (c_lo - c_hi).astype(jnp.float32), np.float32(1.0))
        v_lo = _k2f(lo)
        t_sec = v_lo + (_k2f(hi) - v_lo) * frac
        t_use = jnp.where(has_both, t_sec, t_ref[...])
        t_key = jnp.clip(_f2k(t_use), lo + np.int32(1), hi - np.int32(1))
        c, tf, notdone = probe_update(t_key)
        pdf = np.float32(0.3989423) * jnp.exp(np.float32(-0.5) * tf * tf)
        t_ref[...] = tf + (c - (kf + np.int32(1))).astype(jnp.float32) / (
            np.float32(n) * pdf + np.float32(1e-30))
        return i + np.int32(1), notdone

    jax.lax.while_loop(a_cond, a_body, (np.int32(0), np.int32(1)))

    def b_cond(carry):
        i, notdone = carry
        return jnp.logical_and(i < CAP_B, notdone > 0)

    def b_body(carry):
        i, _ = carry
        lo, hi = lo_ref[...], hi_ref[...]
        t_key = (jax.lax.shift_right_arithmetic(lo, 1)
                 + jax.lax.shift_right_arithmetic(hi, 1)
                 + (lo & hi & np.int32(1)))
        _, _, notdone = probe_update(t_key)
        return i + np.int32(1), notdone

    nd0 = jnp.sum(jnp.logical_not(done_ref[...] > 0).astype(jnp.int32))
    jax.lax.while_loop(b_cond, b_body, (np.int32(0), nd0))

    def e_cond(carry):
        i, anyj = carry
        return jnp.logical_and(i < np.int32(ZONE_M + 2), anyj > 0)

    def e_body(carry):
        i, _ = carry
        j = j_ref[...]
        b_val = _k2f(bkey_ref[...])
        m = jnp.min(jnp.where(x >= b_val, x, np.float32(np.inf)),
                    axis=1, keepdims=True)
        act = j > 0
        bkey_ref[...] = jnp.where(act, _f2k(m) + np.int32(1), bkey_ref[...])
        j2 = jnp.where(act, j - np.int32(1), j)
        j_ref[...] = j2
        return i + np.int32(1), jnp.sum((j2 > 0).astype(jnp.int32))

    aj0 = jnp.sum((j_ref[...] > 0).astype(jnp.int32))
    jax.lax.while_loop(e_cond, e_body, (np.int32(0), aj0))

    o_ref[...] = jnp.where(x >= _k2f(bkey_ref[...]), x, np.float32(0.0))


@jax.jit
def kernel(membrane):
    rows, n = membrane.shape
    k = max(1, int(K_FRAC * n))
    svec_i = pltpu.VMEM((32, 1), jnp.int32)
    block_rows = 32
    return pl.pallas_call(
        functools.partial(_topk_mask_block, k=k, n=n),
        grid=(rows // block_rows,),
        in_specs=[pl.BlockSpec((block_rows, n), lambda i: (i, 0))],
        out_specs=pl.BlockSpec((block_rows, n), lambda i: (i, 0)),
        out_shape=jax.ShapeDtypeStruct((rows, n), membrane.dtype),
        scratch_shapes=[svec_i] * 7 + [pltpu.VMEM((32, 1), jnp.float32)],
        compiler_params=pltpu.CompilerParams(
            dimension_semantics=("arbitrary",),
        ),
    )(membrane)


# speedup vs baseline: 1.1111x; 1.1111x over previous
import functools

import jax
import jax.numpy as jnp
import numpy as np
from jax.experimental import pallas as pl
from jax.experimental.pallas import tpu as pltpu

K_FRAC = 0.1
KEY_NEG_INF = np.int32(-2139095041)
KEY_POS_INF = np.int32(0x7F800000)
ZONE_M = 2
CAP_A = 10
CAP_B = 34
T0 = 1.2815516


def _f2k(f):
    b = jax.lax.bitcast_convert_type(f, jnp.int32)
    return b ^ (jax.lax.shift_right_arithmetic(b, 31) & np.int32(0x7FFFFFFF))


def _k2f(t):
    tc = jnp.maximum(t, KEY_NEG_INF)
    fbits = jnp.where(tc < 0, tc ^ np.int32(0x7FFFFFFF), tc)
    return jax.lax.bitcast_convert_type(fbits, jnp.float32)


def _topk_mask_block(x_ref, o_ref, lo_ref, hi_ref, clo_ref, chi_ref,
                     bkey_ref, j_ref, done_ref, t_ref, *, k, n):
    x = x_ref[...]
    kf = np.int32(k)
    rows = x.shape[0]
    zeros = jnp.zeros((rows, 1), jnp.int32)

    lo_ref[...] = zeros + KEY_NEG_INF
    hi_ref[...] = zeros + KEY_POS_INF
    clo_ref[...] = zeros + np.int32(n)
    chi_ref[...] = zeros
    bkey_ref[...] = zeros
    j_ref[...] = zeros
    done_ref[...] = zeros
    t_ref[...] = jnp.zeros((rows, 1), jnp.float32) + np.float32(T0)

    def count_ge(tf):
        return jnp.sum((x >= tf).astype(jnp.int32), axis=1, keepdims=True)

    def probe_update(t_key):
        tf = _k2f(t_key)
        c = count_ge(tf)
        done = done_ref[...] > 0
        ge = c >= kf
        upd = jnp.logical_not(done)
        lo2 = jnp.where(upd & ge, t_key, lo_ref[...])
        clo2 = jnp.where(upd & ge, c, clo_ref[...])
        hi2 = jnp.where(upd & ~ge, t_key, hi_ref[...])
        chi2 = jnp.where(upd & ~ge, c, chi_ref[...])
        zone = upd & ge & (c <= kf + np.int32(ZONE_M))
        narrow = upd & ~zone & (hi2 <= lo2 + np.int32(1))
        bkey_ref[...] = jnp.where(zone, t_key,
                                  jnp.where(narrow, lo2, bkey_ref[...]))
        j_ref[...] = jnp.where(zone, c - kf,
                               jnp.where(narrow, np.int32(0), j_ref[...]))
        done2 = done | zone | narrow
        lo_ref[...] = lo2
        hi_ref[...] = hi2
        clo_ref[...] = clo2
        chi_ref[...] = chi2
        done_ref[...] = done2.astype(jnp.int32)
        notdone = jnp.sum(jnp.logical_not(done2).astype(jnp.int32))
        return c, tf, notdone

    def a_cond(carry):
        i, notdone = carry
        return jnp.logical_and(i < CAP_A, notdone > 0)

    def a_body(carry):
        i, _ = carry
        lo, hi = lo_ref[...], hi_ref[...]
        c_lo, c_hi = clo_ref[...], chi_ref[...]
        has_both = (lo > KEY_NEG_INF) & (hi < KEY_POS_INF)
        frac = (c_lo - (kf + np.int32(1))).astype(jnp.float32) / jnp.maximum(
            (c_lo - c_hi).astype(jnp.float32), np.float32(1.0))
        v_lo = _k2f(lo)
        t_sec = v_lo + (_k2f(hi) - v_lo) * frac
        t_use = jnp.where(has_both, t_sec, t_ref[...])
        t_key = jnp.clip(_f2k(t_use), lo + np.int32(1), hi - np.int32(1))
        c, tf, notdone = probe_update(t_key)
        pdf = np.float32(0.3989423) * jnp.exp(np.float32(-0.5) * tf * tf)
        t_ref[...] = tf + (c - (kf + np.int32(1))).astype(jnp.float32) / (
            np.float32(n) * pdf + np.float32(1e-30))
        return i + np.int32(1), notdone

    jax.lax.while_loop(a_cond, a_body, (np.int32(0), np.int32(1)))

    def b_cond(carry):
        i, notdone = carry
        return jnp.logical_and(i < CAP_B, notdone > 0)

    def b_body(carry):
        i, _ = carry
        lo, hi = lo_ref[...], hi_ref[...]
        t_key = (jax.lax.shift_right_arithmetic(lo, 1)
                 + jax.lax.shift_right_arithmetic(hi, 1)
                 + (lo & hi & np.int32(1)))
        _, _, notdone = probe_update(t_key)
        return i + np.int32(1), notdone

    nd0 = jnp.sum(jnp.logical_not(done_ref[...] > 0).astype(jnp.int32))
    jax.lax.while_loop(b_cond, b_body, (np.int32(0), nd0))

    def e_cond(carry):
        i, anyj = carry
        return jnp.logical_and(i < np.int32(ZONE_M + 2), anyj > 0)

    def e_body(carry):
        i, _ = carry
        j = j_ref[...]
        b_val = _k2f(bkey_ref[...])
        m = jnp.min(jnp.where(x >= b_val, x, np.float32(np.inf)),
                    axis=1, keepdims=True)
        act = j > 0
        bkey_ref[...] = jnp.where(act, _f2k(m) + np.int32(1), bkey_ref[...])
        j2 = jnp.where(act, j - np.int32(1), j)
        j_ref[...] = j2
        return i + np.int32(1), jnp.sum((j2 > 0).astype(jnp.int32))

    aj0 = jnp.sum((j_ref[...] > 0).astype(jnp.int32))
    jax.lax.while_loop(e_cond, e_body, (np.int32(0), aj0))

    o_ref[...] = jnp.where(x >= _k2f(bkey_ref[...]), x, np.float32(0.0))


@jax.jit
def kernel(membrane):
    rows, n = membrane.shape
    k = max(1, int(K_FRAC * n))
    svec_i = pltpu.VMEM((64, 1), jnp.int32)
    block_rows = 64
    return pl.pallas_call(
        functools.partial(_topk_mask_block, k=k, n=n),
        grid=(rows // block_rows,),
        in_specs=[pl.BlockSpec((block_rows, n), lambda i: (i, 0))],
        out_specs=pl.BlockSpec((block_rows, n), lambda i: (i, 0)),
        out_shape=jax.ShapeDtypeStruct((rows, n), membrane.dtype),
        scratch_shapes=[svec_i] * 7 + [pltpu.VMEM((64, 1), jnp.float32)],
        compiler_params=pltpu.CompilerParams(
            dimension_semantics=("arbitrary",),
        ),
    )(membrane)
